# Initial kernel scaffold; baseline (speedup 1.0000x reference)
import jax
import jax.numpy as jnp
from jax.experimental import pallas as pl


def kernel(pose_table, exp_table, cam_table, light_table, full_pose, frame_ids):
    raise NotImplementedError("write your pallas kernel here")


# speedup vs baseline: 1.6141x; 1.6141x over previous
import functools
import math

import jax
import jax.numpy as jnp
from jax import lax
from jax.experimental import pallas as pl
from jax.experimental.pallas import tpu as pltpu
from jax.experimental.pallas import tpu_sc as plsc

_N_JOINTS = 55
_EXP_DIM = 100
_PW = 168
_EW = 104
_CW = 8
_LW = 32
_FW = _N_JOINTS * 9
_FREE_JOINTS = (0, 12, 15, 16, 17, 22, 23, 24)

_COS_C = tuple((-1.0) ** k / math.factorial(2 * k) for k in range(8))
_SINC_C = tuple((-1.0) ** k / math.factorial(2 * k + 1) for k in range(7))
_VERC_C = tuple((-1.0) ** k / math.factorial(2 * k + 2) for k in range(7))


def _horner(u, coeffs):
    acc = jnp.full((16,), coeffs[-1], jnp.float32)
    for c in coeffs[-2::-1]:
        acc = acc * u + c
    return acc


def _sc_body(nc, bpw,
             pose_hbm, exp_hbm, cam_hbm, light_hbm, fp_hbm, ids_hbm,
             out_pose, out_cam, out_exp, out_light,
             idx_v, pose_v, exp_v, cam_v, light_v, fp_v,
             sem_fp, sem_p, sem_e, sem_c, sem_l):
    wid = lax.axis_index("s") * nc + lax.axis_index("c")
    base = wid * bpw

    fp_cp = pltpu.async_copy(fp_hbm.at[pl.ds(base, bpw)], fp_v, sem_fp)
    pltpu.sync_copy(ids_hbm.at[pl.ds(base, bpw)], idx_v)
    g_pose = pltpu.async_copy(pose_hbm.at[idx_v], pose_v, sem_p)
    g_exp = pltpu.async_copy(exp_hbm.at[idx_v], exp_v, sem_e)
    g_cam = pltpu.async_copy(cam_hbm.at[idx_v], cam_v, sem_c)
    g_light = pltpu.async_copy(light_hbm.at[idx_v], light_v, sem_l)

    g_exp.wait()
    pltpu.sync_copy(exp_v, out_exp.at[pl.ds(base, bpw)])
    g_cam.wait()
    pltpu.sync_copy(cam_v, out_cam.at[pl.ds(base, bpw)])
    g_light.wait()
    pltpu.sync_copy(light_v, out_light.at[pl.ds(base, bpw)])
    g_pose.wait()
    fp_cp.wait()

    lanes = lax.iota(jnp.int32, 16)

    def group(g, carry):
        frames = g * 16 + lanes
        for j in _FREE_JOINTS:
            x = plsc.load_gather(pose_v, [frames, jnp.full((16,), 3 * j, jnp.int32)])
            y = plsc.load_gather(pose_v, [frames, jnp.full((16,), 3 * j + 1, jnp.int32)])
            z = plsc.load_gather(pose_v, [frames, jnp.full((16,), 3 * j + 2, jnp.int32)])
            u = x * x + y * y + z * z
            c = _horner(u, _COS_C)
            a = _horner(u, _SINC_C)
            v = _horner(u, _VERC_C)
            ax, ay, az = a * x, a * y, a * z
            vx, vy, vz = v * x, v * y, v * z
            ent = (
                c + vx * x, vx * y - az, vx * z + ay,
                vx * y + az, c + vy * y, vy * z - ax,
                vx * z - ay, vy * z + ax, c + vz * z,
            )
            for e, val in enumerate(ent):
                plsc.store_scatter(
                    fp_v, [frames, jnp.full((16,), 9 * j + e, jnp.int32)], val)
        return carry

    lax.fori_loop(0, bpw // 16, group, 0)
    pltpu.sync_copy(fp_v, out_pose.at[pl.ds(base, bpw)])


def kernel(pose_table, exp_table, cam_table, light_table, full_pose, frame_ids):
    b = frame_ids.shape[0]
    n_frames = pose_table.shape[0]
    info = plsc.get_sparse_core_info()
    nc, ns = info.num_cores, info.num_subcores
    nw = nc * ns
    bpw = b // nw

    pose2d = jnp.pad(pose_table.reshape(n_frames, _N_JOINTS * 3),
                     ((0, 0), (0, _PW - _N_JOINTS * 3)))
    exp2d = jnp.pad(exp_table, ((0, 0), (0, _EW - _EXP_DIM)))
    cam2d = jnp.pad(cam_table, ((0, 0), (0, _CW - 3)))
    light2d = jnp.pad(light_table.reshape(n_frames, 27), ((0, 0), (0, _LW - 27)))
    fp2d = full_pose.reshape(b, _FW)

    mesh = plsc.VectorSubcoreMesh(core_axis_name="c", subcore_axis_name="s")
    run = functools.partial(
        pl.kernel,
        out_type=(
            jax.ShapeDtypeStruct((b, _FW), jnp.float32),
            jax.ShapeDtypeStruct((b, _CW), jnp.float32),
            jax.ShapeDtypeStruct((b, _EW), jnp.float32),
            jax.ShapeDtypeStruct((b, _LW), jnp.float32),
        ),
        mesh=mesh,
        compiler_params=pltpu.CompilerParams(
            needs_layout_passes=False, use_tc_tiling_on_sc=False),
        scratch_types=(
            pltpu.VMEM((bpw,), jnp.int32),
            pltpu.VMEM((bpw, _PW), jnp.float32),
            pltpu.VMEM((bpw, _EW), jnp.float32),
            pltpu.VMEM((bpw, _CW), jnp.float32),
            pltpu.VMEM((bpw, _LW), jnp.float32),
            pltpu.VMEM((bpw, _FW), jnp.float32),
            pltpu.SemaphoreType.DMA,
            pltpu.SemaphoreType.DMA,
            pltpu.SemaphoreType.DMA,
            pltpu.SemaphoreType.DMA,
            pltpu.SemaphoreType.DMA,
        ),
    )(functools.partial(_sc_body, nc, bpw))

    out_pose, cam, exp, light = run(
        pose2d, exp2d, cam2d, light2d, fp2d, frame_ids.astype(jnp.int32))
    return (out_pose.reshape(b, _N_JOINTS, 3, 3), cam[:, :3], exp[:, :_EXP_DIM],
            light[:, :27].reshape(b, 9, 3))


# speedup vs baseline: 3.8373x; 2.3774x over previous
import functools
import math

import jax
import jax.numpy as jnp
from jax import lax
from jax.experimental import pallas as pl
from jax.experimental.pallas import tpu as pltpu
from jax.experimental.pallas import tpu_sc as plsc

_N_JOINTS = 55
_EXP_DIM = 100
_PW = 24
_EW = 104
_CW = 8
_LW = 32
_FW = _N_JOINTS * 9
_FREE_JOINTS = (0, 12, 15, 16, 17, 22, 23, 24)

_COS_C = tuple((-1.0) ** k / math.factorial(2 * k) for k in range(8))
_SINC_C = tuple((-1.0) ** k / math.factorial(2 * k + 1) for k in range(7))
_VERC_C = tuple((-1.0) ** k / math.factorial(2 * k + 2) for k in range(7))


def _horner(u, coeffs):
    acc = jnp.full((16,), coeffs[-1], jnp.float32)
    for c in coeffs[-2::-1]:
        acc = acc * u + c
    return acc


def _sc_body(nc, bpw,
             pose_hbm, exp_hbm, cam_hbm, light_hbm, fp_hbm, ids_hbm,
             out_pose, out_cam, out_exp, out_light,
             idx_v, pose_v, exp_v, cam_v, light_v, fp_v,
             sem_fp, sem_p, sem_e, sem_c, sem_l):
    wid = lax.axis_index("s") * nc + lax.axis_index("c")
    base = wid * bpw

    fp_cp = pltpu.async_copy(fp_hbm.at[pl.ds(base, bpw)], fp_v, sem_fp)
    pltpu.sync_copy(ids_hbm.at[pl.ds(base, bpw)], idx_v)
    g_pose = pltpu.async_copy(pose_hbm.at[idx_v], pose_v, sem_p)
    g_exp = pltpu.async_copy(exp_hbm.at[idx_v], exp_v, sem_e)
    g_cam = pltpu.async_copy(cam_hbm.at[idx_v], cam_v, sem_c)
    g_light = pltpu.async_copy(light_hbm.at[idx_v], light_v, sem_l)

    g_exp.wait()
    pltpu.sync_copy(exp_v, out_exp.at[pl.ds(base, bpw)])
    g_cam.wait()
    pltpu.sync_copy(cam_v, out_cam.at[pl.ds(base, bpw)])
    g_light.wait()
    pltpu.sync_copy(light_v, out_light.at[pl.ds(base, bpw)])
    g_pose.wait()
    fp_cp.wait()

    lanes = lax.iota(jnp.int32, 16)

    def group(g, carry):
        frames = g * 16 + lanes
        for k, j in enumerate(_FREE_JOINTS):
            x = plsc.load_gather(pose_v, [frames, jnp.full((16,), 3 * k, jnp.int32)])
            y = plsc.load_gather(pose_v, [frames, jnp.full((16,), 3 * k + 1, jnp.int32)])
            z = plsc.load_gather(pose_v, [frames, jnp.full((16,), 3 * k + 2, jnp.int32)])
            u = x * x + y * y + z * z
            c = _horner(u, _COS_C)
            a = _horner(u, _SINC_C)
            v = _horner(u, _VERC_C)
            ax, ay, az = a * x, a * y, a * z
            vx, vy, vz = v * x, v * y, v * z
            ent = (
                c + vx * x, vx * y - az, vx * z + ay,
                vx * y + az, c + vy * y, vy * z - ax,
                vx * z - ay, vy * z + ax, c + vz * z,
            )
            for e, val in enumerate(ent):
                plsc.store_scatter(
                    fp_v, [frames, jnp.full((16,), 9 * j + e, jnp.int32)], val)
        return carry

    lax.fori_loop(0, bpw // 16, group, 0)
    pltpu.sync_copy(fp_v, out_pose.at[pl.ds(base, bpw)])


def kernel(pose_table, exp_table, cam_table, light_table, full_pose, frame_ids):
    b = frame_ids.shape[0]
    n_frames = pose_table.shape[0]
    info = plsc.get_sparse_core_info()
    nc, ns = info.num_cores, info.num_subcores
    nw = nc * ns
    bpw = b // nw

    pose2d = pose_table[:, _FREE_JOINTS, :].reshape(n_frames, _PW)
    exp2d = jnp.pad(exp_table, ((0, 0), (0, _EW - _EXP_DIM)))
    cam2d = jnp.pad(cam_table, ((0, 0), (0, _CW - 3)))
    light2d = jnp.pad(light_table.reshape(n_frames, 27), ((0, 0), (0, _LW - 27)))
    fp2d = full_pose.reshape(b, _FW)

    mesh = plsc.VectorSubcoreMesh(core_axis_name="c", subcore_axis_name="s")
    run = functools.partial(
        pl.kernel,
        out_type=(
            jax.ShapeDtypeStruct((b, _FW), jnp.float32),
            jax.ShapeDtypeStruct((b, _CW), jnp.float32),
            jax.ShapeDtypeStruct((b, _EW), jnp.float32),
            jax.ShapeDtypeStruct((b, _LW), jnp.float32),
        ),
        mesh=mesh,
        compiler_params=pltpu.CompilerParams(
            needs_layout_passes=False, use_tc_tiling_on_sc=False),
        scratch_types=(
            pltpu.VMEM((bpw,), jnp.int32),
            pltpu.VMEM((bpw, _PW), jnp.float32),
            pltpu.VMEM((bpw, _EW), jnp.float32),
            pltpu.VMEM((bpw, _CW), jnp.float32),
            pltpu.VMEM((bpw, _LW), jnp.float32),
            pltpu.VMEM((bpw, _FW), jnp.float32),
            pltpu.SemaphoreType.DMA,
            pltpu.SemaphoreType.DMA,
            pltpu.SemaphoreType.DMA,
            pltpu.SemaphoreType.DMA,
            pltpu.SemaphoreType.DMA,
        ),
    )(functools.partial(_sc_body, nc, bpw))

    out_pose, cam, exp, light = run(
        pose2d, exp2d, cam2d, light2d, fp2d, frame_ids.astype(jnp.int32))
    return (out_pose.reshape(b, _N_JOINTS, 3, 3), cam[:, :3], exp[:, :_EXP_DIM],
            light[:, :27].reshape(b, 9, 3))


# speedup vs baseline: 4.4931x; 1.1709x over previous
import functools
import math

import jax
import jax.numpy as jnp
from jax import lax
from jax.experimental import pallas as pl
from jax.experimental.pallas import tpu as pltpu
from jax.experimental.pallas import tpu_sc as plsc

_N_JOINTS = 55
_EXP_DIM = 100
_PW = 24
_EW = 104
_CW = 8
_LW = 32
_FREE_JOINTS = (0, 12, 15, 16, 17, 22, 23, 24)

_COS_C = tuple((-1.0) ** k / math.factorial(2 * k) for k in range(8))
_SINC_C = tuple((-1.0) ** k / math.factorial(2 * k + 1) for k in range(7))
_VERC_C = tuple((-1.0) ** k / math.factorial(2 * k + 2) for k in range(7))


def _horner(u, coeffs):
    acc = jnp.full((16,), coeffs[-1], jnp.float32)
    for c in coeffs[-2::-1]:
        acc = acc * u + c
    return acc


def _sc_body(nc, bpw,
             pose_hbm, exp_hbm, cam_hbm, light_hbm, ids_hbm,
             out_mats, out_cam, out_exp, out_light,
             idx_v, pose_v, exp_v, cam_v, light_v, mats_v,
             sem_p, sem_e, sem_c, sem_l):
    wid = lax.axis_index("s") * nc + lax.axis_index("c")
    base = wid * bpw

    pltpu.sync_copy(ids_hbm.at[pl.ds(base, bpw)], idx_v)
    g_pose = pltpu.async_copy(pose_hbm.at[idx_v], pose_v, sem_p)
    g_exp = pltpu.async_copy(exp_hbm.at[idx_v], exp_v, sem_e)
    g_cam = pltpu.async_copy(cam_hbm.at[idx_v], cam_v, sem_c)
    g_light = pltpu.async_copy(light_hbm.at[idx_v], light_v, sem_l)

    g_pose.wait()
    lanes = lax.iota(jnp.int32, 16)

    def group(g, carry):
        frames = g * 16 + lanes
        for k in range(len(_FREE_JOINTS)):
            x = plsc.load_gather(pose_v, [frames, jnp.full((16,), 3 * k, jnp.int32)])
            y = plsc.load_gather(pose_v, [frames, jnp.full((16,), 3 * k + 1, jnp.int32)])
            z = plsc.load_gather(pose_v, [frames, jnp.full((16,), 3 * k + 2, jnp.int32)])
            u = x * x + y * y + z * z
            c = _horner(u, _COS_C)
            a = _horner(u, _SINC_C)
            v = _horner(u, _VERC_C)
            ax, ay, az = a * x, a * y, a * z
            vx, vy, vz = v * x, v * y, v * z
            ent = (
                c + vx * x, vx * y - az, vx * z + ay,
                vx * y + az, c + vy * y, vy * z - ax,
                vx * z - ay, vy * z + ax, c + vz * z,
            )
            for e, val in enumerate(ent):
                mats_v[k * 9 + e, pl.ds(g * 16, 16)] = val
        return carry

    lax.fori_loop(0, bpw // 16, group, 0)
    pltpu.sync_copy(mats_v, out_mats.at[wid])

    g_exp.wait()
    pltpu.sync_copy(exp_v, out_exp.at[pl.ds(base, bpw)])
    g_cam.wait()
    pltpu.sync_copy(cam_v, out_cam.at[pl.ds(base, bpw)])
    g_light.wait()
    pltpu.sync_copy(light_v, out_light.at[pl.ds(base, bpw)])


def _tc_merge_body(fp_ref, mats_ref, out_ref):
    out_ref[...] = fp_ref[...]
    for k, j in enumerate(_FREE_JOINTS):
        for e in range(9):
            out_ref[e, j, :] = mats_ref[0, k * 9 + e, :]


def kernel(pose_table, exp_table, cam_table, light_table, full_pose, frame_ids):
    b = frame_ids.shape[0]
    n_frames = pose_table.shape[0]
    info = plsc.get_sparse_core_info()
    nc, ns = info.num_cores, info.num_subcores
    nw = nc * ns
    bpw = b // nw

    pose2d = pose_table[:, _FREE_JOINTS, :].reshape(n_frames, _PW)
    exp2d = jnp.pad(exp_table, ((0, 0), (0, _EW - _EXP_DIM)))
    cam2d = jnp.pad(cam_table, ((0, 0), (0, _CW - 3)))
    light2d = jnp.pad(light_table.reshape(n_frames, 27), ((0, 0), (0, _LW - 27)))

    mesh = plsc.VectorSubcoreMesh(core_axis_name="c", subcore_axis_name="s")
    run = functools.partial(
        pl.kernel,
        out_type=(
            jax.ShapeDtypeStruct((nw, 72, bpw), jnp.float32),
            jax.ShapeDtypeStruct((b, _CW), jnp.float32),
            jax.ShapeDtypeStruct((b, _EW), jnp.float32),
            jax.ShapeDtypeStruct((b, _LW), jnp.float32),
        ),
        mesh=mesh,
        compiler_params=pltpu.CompilerParams(
            needs_layout_passes=False, use_tc_tiling_on_sc=False),
        scratch_types=(
            pltpu.VMEM((bpw,), jnp.int32),
            pltpu.VMEM((bpw, _PW), jnp.float32),
            pltpu.VMEM((bpw, _EW), jnp.float32),
            pltpu.VMEM((bpw, _CW), jnp.float32),
            pltpu.VMEM((bpw, _LW), jnp.float32),
            pltpu.VMEM((72, bpw), jnp.float32),
            pltpu.SemaphoreType.DMA,
            pltpu.SemaphoreType.DMA,
            pltpu.SemaphoreType.DMA,
            pltpu.SemaphoreType.DMA,
        ),
    )(functools.partial(_sc_body, nc, bpw))

    mats, cam, exp, light = run(
        pose2d, exp2d, cam2d, light2d, frame_ids.astype(jnp.int32))

    fp_t = jnp.transpose(full_pose, (2, 3, 1, 0)).reshape(9, _N_JOINTS, b)
    out_t = pl.pallas_call(
        _tc_merge_body,
        grid=(nw,),
        in_specs=[
            pl.BlockSpec((9, _N_JOINTS, bpw), lambda i: (0, 0, i)),
            pl.BlockSpec((1, 72, bpw), lambda i: (i, 0, 0)),
        ],
        out_specs=pl.BlockSpec((9, _N_JOINTS, bpw), lambda i: (0, 0, i)),
        out_shape=jax.ShapeDtypeStruct((9, _N_JOINTS, b), jnp.float32),
    )(fp_t, mats)
    out_pose = jnp.transpose(
        out_t.reshape(3, 3, _N_JOINTS, b), (3, 2, 0, 1))

    return (out_pose, cam[:, :3], exp[:, :_EXP_DIM],
            light[:, :27].reshape(b, 9, 3))
